# Initial kernel scaffold; baseline (speedup 1.0000x reference)
import jax
import jax.numpy as jnp
from jax.experimental import pallas as pl


def kernel(x, edge_index, c1w0, c1b0, c1w1, c1b1, c1w2, c1b2, c2w0, c2b0, c2w1, c2b1, c2w2, c2b2, lw, lb):
    raise NotImplementedError("write your pallas kernel here")


# speedup vs baseline: 6.2225x; 6.2225x over previous
import functools

import jax
import jax.numpy as jnp
from jax import lax
from jax.experimental import pallas as pl
from jax.experimental.pallas import tpu as pltpu
from jax.experimental.pallas import tpu_sc as plsc

N = 10000
NPAD = 10240
E = 320000
EPAD = 327680
EPB = EPAD // 128
IN_DIM = 128
HID = 256
OUT_DIM = 128
NTILES = 16
ROWS_PT = NPAD // NTILES
BN = 256

_MESH = dict(mesh=plsc.VectorSubcoreMesh(core_axis_name="c", subcore_axis_name="s"))


def _rsqrt16(x):
    y = 1.0 / x
    for _ in range(26):
        y = y * (1.5 - 0.5 * x * y * y)
    return y


@functools.partial(
    pl.kernel,
    out_type=(
        jax.ShapeDtypeStruct((NPAD,), jnp.float32),
        jax.ShapeDtypeStruct((NPAD, IN_DIM), jnp.float32),
    ),
    scratch_types=[
        pltpu.VMEM_SHARED((NPAD,), jnp.float32),
        pltpu.VMEM((ROWS_PT,), jnp.float32),
        pltpu.VMEM((128,), jnp.float32),
        pltpu.VMEM((1, 128), jnp.int32),
        pltpu.VMEM((64, IN_DIM), jnp.float32),
        pltpu.VMEM((64, IN_DIM), jnp.float32),
    ],
    **_MESH,
)
def _sc0(dst_hbm, x_hbm, dinv_hbm, y0_hbm, hist, dvb, ones, dbuf, xbuf, ybuf):
    c = lax.axis_index("c")
    s = lax.axis_index("s")

    def fz(i, _):
        dvb[pl.ds(i * 16, 16)] = jnp.zeros((16,), jnp.float32)
        return 0
    lax.fori_loop(0, ROWS_PT // 16, fz, 0)
    for k in range(8):
        ones[pl.ds(k * 16, 16)] = jnp.ones((16,), jnp.float32)
    pltpu.sync_copy(dvb, hist.at[pl.ds(s * ROWS_PT, ROWS_PT)])
    plsc.subcore_barrier()

    blks_pt = EPB // NTILES
    base = s * blks_pt

    def fa(b, _):
        pltpu.sync_copy(dst_hbm.at[pl.ds(base + b, 1)], dbuf)
        pltpu.sync_copy(ones, hist.at[dbuf.at[0]], add=True)
        return 0
    lax.fori_loop(0, blks_pt, fa, 0)
    plsc.subcore_barrier()

    pltpu.sync_copy(hist.at[pl.ds(s * ROWS_PT, ROWS_PT)], dvb)

    def fb(j, _):
        d = dvb[pl.ds(j * 16, 16)] + 1.0
        dvb[pl.ds(j * 16, 16)] = _rsqrt16(d)
        return 0
    lax.fori_loop(0, ROWS_PT // 16, fb, 0)

    @pl.when(c == 0)
    def _():
        pltpu.sync_copy(dvb, dinv_hbm.at[pl.ds(s * ROWS_PT, ROWS_PT)])

    def fc(i, _):
        r = s * ROWS_PT + c * 320 + i * 64
        pltpu.sync_copy(x_hbm.at[pl.ds(r, 64)], xbuf)
        for j16 in range(4):
            dv16 = dvb[pl.ds(c * 320 + i * 64 + j16 * 16, 16)]
            for l in range(16):
                j = j16 * 16 + l
                dv = dv16[l]
                for k in range(IN_DIM // 16):
                    ybuf[j, pl.ds(k * 16, 16)] = xbuf[j, pl.ds(k * 16, 16)] * dv
        pltpu.sync_copy(ybuf, y0_hbm.at[pl.ds(r, 64)])
        return 0
    lax.fori_loop(0, 5, fc, 0)


@functools.partial(
    pl.kernel,
    out_type=jax.ShapeDtypeStruct((2, NPAD, 128), jnp.float32),
    scratch_types=[
        pltpu.VMEM_SHARED((NPAD, 128), jnp.float32),
        pltpu.VMEM((64, 128), jnp.float32),
        pltpu.VMEM((1, 128), jnp.int32),
        pltpu.VMEM((1, 128), jnp.int32),
        pltpu.VMEM((128, 128), jnp.float32),
        pltpu.SemaphoreType.DMA,
    ],
    **_MESH,
)
def _prop_part(y_hbm, src_hbm, dst_hbm, out_hbm, acc, abuf, sbuf, dbuf, rows,
               gsem):
    c = lax.axis_index("c")
    s = lax.axis_index("s")

    for j in range(64):
        for k in range(8):
            abuf[j, pl.ds(k * 16, 16)] = jnp.zeros((16,), jnp.float32)

    def fzc(i, _):
        pltpu.sync_copy(abuf, acc.at[pl.ds(s * ROWS_PT + i * 64, 64)])
        return 0
    lax.fori_loop(0, 10, fzc, 0)
    plsc.subcore_barrier()

    blks_pt = EPB // 32
    base = (c * NTILES + s) * blks_pt

    def fe(b, _):
        pltpu.sync_copy(src_hbm.at[pl.ds(base + b, 1)], sbuf)
        pltpu.sync_copy(dst_hbm.at[pl.ds(base + b, 1)], dbuf)
        pltpu.async_copy(y_hbm.at[sbuf.at[0]], rows, gsem).wait()
        pltpu.sync_copy(rows, acc.at[dbuf.at[0]], add=True)
        return 0
    lax.fori_loop(0, blks_pt, fe, 0)
    plsc.subcore_barrier()

    def fo(i, _):
        r = s * ROWS_PT + i * 64
        pltpu.sync_copy(acc.at[pl.ds(r, 64)], abuf)
        pltpu.sync_copy(abuf, out_hbm.at[c, pl.ds(r, 64)])
        return 0
    lax.fori_loop(0, 10, fo, 0)


def _make_prop(emit_y):
    one = jax.ShapeDtypeStruct((2 * NPAD, 128), jnp.float32)

    @functools.partial(
        pl.kernel,
        out_type=(one, one) if emit_y else one,
        scratch_types=[
            pltpu.VMEM_SHARED((NPAD, 128), jnp.float32),
            pltpu.VMEM((32, 128), jnp.float32),
            pltpu.VMEM((32, 128), jnp.float32),
            pltpu.VMEM((32, 128), jnp.float32),
            pltpu.VMEM((32, 128), jnp.float32),
            pltpu.VMEM((1, 128), jnp.int32),
            pltpu.VMEM((1, 128), jnp.int32),
            pltpu.VMEM((1, 128), jnp.int32),
            pltpu.VMEM((128, 128), jnp.float32),
            pltpu.VMEM((ROWS_PT,), jnp.float32),
            pltpu.SemaphoreType.DMA,
        ],
        **_MESH,
    )
    def prop(y_hbm, src_hbm, dst_hbm, dinv_hbm, *rest):
        if emit_y:
            out_hbm, yout_hbm = rest[0], rest[1]
            rest = rest[2:]
        else:
            out_hbm, yout_hbm = rest[0], None
            rest = rest[1:]
        acc, abuf, ybuf, obuf, y2buf, sbuf, sidx, dbuf, rows, dvb, gsem = rest

        c = lax.axis_index("c")
        s = lax.axis_index("s")
        coff = c * NPAD

        for j in range(32):
            for k in range(8):
                abuf[j, pl.ds(k * 16, 16)] = jnp.zeros((16,), jnp.float32)

        def fzc(i, _):
            pltpu.sync_copy(abuf, acc.at[pl.ds(s * ROWS_PT + i * 32, 32)])
            return 0
        lax.fori_loop(0, 20, fzc, 0)
        plsc.subcore_barrier()

        blks_pt = EPB // NTILES
        base = s * blks_pt

        def fe(b, _):
            pltpu.sync_copy(src_hbm.at[pl.ds(base + b, 1)], sbuf)
            pltpu.sync_copy(dst_hbm.at[pl.ds(base + b, 1)], dbuf)
            for k in range(8):
                sidx[0, pl.ds(k * 16, 16)] = sbuf[0, pl.ds(k * 16, 16)] + coff
            pltpu.async_copy(y_hbm.at[sidx.at[0]], rows, gsem).wait()
            pltpu.sync_copy(rows, acc.at[dbuf.at[0]], add=True)
            return 0
        lax.fori_loop(0, blks_pt, fe, 0)
        plsc.subcore_barrier()

        pltpu.sync_copy(dinv_hbm.at[pl.ds(s * ROWS_PT, ROWS_PT)], dvb)

        def fo(i, _):
            r = s * ROWS_PT + i * 32
            pltpu.sync_copy(acc.at[pl.ds(r, 32)], abuf)
            pltpu.sync_copy(y_hbm.at[pl.ds(coff + r, 32)], ybuf)
            for j16 in range(2):
                dv16 = dvb[pl.ds(i * 32 + j16 * 16, 16)]
                for l in range(16):
                    j = j16 * 16 + l
                    dv = dv16[l]
                    for k in range(8):
                        o = (abuf[j, pl.ds(k * 16, 16)]
                             + ybuf[j, pl.ds(k * 16, 16)]) * dv
                        obuf[j, pl.ds(k * 16, 16)] = o
                        if emit_y:
                            y2buf[j, pl.ds(k * 16, 16)] = o * dv
            pltpu.sync_copy(obuf, out_hbm.at[pl.ds(coff + r, 32)])
            if emit_y:
                pltpu.sync_copy(y2buf, yout_hbm.at[pl.ds(coff + r, 32)])
            return 0
        lax.fori_loop(0, 20, fo, 0)

    return prop


_prop128y = _make_prop(True)
_prop128 = _make_prop(False)


def _combine_body(pa_ref, pb_ref, y_ref, dv_ref, p_ref, yn_ref):
    dv = dv_ref[...][:, 0:1]
    p = dv * (pa_ref[...] + pb_ref[...] + y_ref[...])
    p_ref[...] = p
    yn_ref[...] = dv * p


def _tc1_body(x_ref, p1_ref, p2_ref, dv_ref, w0, b0, w1, b1, w2, b2,
              wc0, bc0, wc1, wc2, u0_ref, yu1_ref, yu2_ref):
    f32 = jnp.float32
    h0 = jnp.dot(x_ref[...], w0[...], preferred_element_type=f32) + b0[...]
    h1 = jnp.dot(p1_ref[...], w1[...], preferred_element_type=f32) + b1[...]
    h2 = jnp.dot(p2_ref[...], w2[...], preferred_element_type=f32) + b2[...]
    hb = jax.nn.relu(jnp.concatenate([h0, h1, h2], axis=-1))
    dv = dv_ref[...][:, 0:1]
    u0_ref[...] = jnp.dot(hb, wc0[...], preferred_element_type=f32) + bc0[...]
    yu1_ref[...] = jnp.dot(hb, wc1[...], preferred_element_type=f32) * dv
    yu2_ref[...] = jnp.dot(hb, wc2[...], preferred_element_type=f32) * dv


def _tc2_body(u0_ref, q1_ref, q2_ref, bc1, bc2, lw, lb, o_ref):
    h2 = jax.nn.relu(jnp.concatenate(
        [u0_ref[...], q1_ref[...] + bc1[...], q2_ref[...] + bc2[...]], axis=-1))
    o_ref[...] = jnp.dot(h2, lw[...], preferred_element_type=jnp.float32) + lb[...]


def _full(shape):
    return pl.BlockSpec(shape, lambda i: tuple(0 for _ in shape))


def _rows(d):
    return pl.BlockSpec((BN, d), lambda i: (i, 0))


_combine = pl.pallas_call(
    _combine_body,
    grid=(NPAD // BN,),
    in_specs=[_rows(128), _rows(128), _rows(128), _rows(128)],
    out_specs=[_rows(128), _rows(128)],
    out_shape=[
        jax.ShapeDtypeStruct((NPAD, 128), jnp.float32),
        jax.ShapeDtypeStruct((NPAD, 128), jnp.float32),
    ],
)

_tc1 = pl.pallas_call(
    _tc1_body,
    grid=(NPAD // BN,),
    in_specs=[
        _rows(128), _rows(128), _rows(128), _rows(128),
        _full((IN_DIM, HID)), _full((1, HID)),
        _full((IN_DIM, HID)), _full((1, HID)),
        _full((IN_DIM, HID)), _full((1, HID)),
        _full((3 * HID, HID)), _full((1, HID)),
        _full((3 * HID, HID)), _full((3 * HID, HID)),
    ],
    out_specs=[_rows(HID), _rows(HID), _rows(HID)],
    out_shape=[
        jax.ShapeDtypeStruct((NPAD, HID), jnp.float32),
        jax.ShapeDtypeStruct((NPAD, HID), jnp.float32),
        jax.ShapeDtypeStruct((NPAD, HID), jnp.float32),
    ],
)

_tc2 = pl.pallas_call(
    _tc2_body,
    grid=(NPAD // BN,),
    in_specs=[
        _rows(HID), _rows(HID), _rows(HID),
        _full((1, HID)), _full((1, HID)),
        _full((3 * HID, OUT_DIM)), _full((1, OUT_DIM)),
    ],
    out_specs=_rows(OUT_DIM),
    out_shape=jax.ShapeDtypeStruct((NPAD, OUT_DIM), jnp.float32),
)


def _to_chunks(a):
    return a.reshape(NPAD, 2, 128).transpose(1, 0, 2).reshape(2 * NPAD, 128)


def _from_chunks(a):
    return a.reshape(2, NPAD, 128).transpose(1, 0, 2).reshape(NPAD, 256)


def kernel(x, edge_index, c1w0, c1b0, c1w1, c1b1, c1w2, c1b2,
           c2w0, c2b0, c2w1, c2b1, c2w2, c2b2, lw, lb):
    xp = jnp.pad(x, ((0, NPAD - N), (0, 0)))
    src = jnp.pad(edge_index[0].astype(jnp.int32), (0, EPAD - E)).reshape(EPB, 128)
    dst = jnp.pad(edge_index[1].astype(jnp.int32), (0, EPAD - E),
                  constant_values=N).reshape(EPB, 128)

    dinv, y0 = _sc0(dst, xp)
    dv2d = jnp.broadcast_to(dinv[:, None], (NPAD, 128))

    pp = _prop_part(y0, src, dst)
    p1, y1 = _combine(pp[0], pp[1], y0, dv2d)
    pp2 = _prop_part(y1, src, dst)
    p2, _ = _combine(pp2[0], pp2[1], y1, dv2d)

    u0, yu1, yu2 = _tc1(
        xp, p1, p2, dv2d,
        c1w0, c1b0.reshape(1, HID), c1w1, c1b1.reshape(1, HID),
        c1w2, c1b2.reshape(1, HID),
        c2w0, c2b0.reshape(1, HID), c2w1, c2w2,
    )

    q1 = _prop128(_to_chunks(yu1), src, dst, dinv)
    _t, yt = _prop128y(_to_chunks(yu2), src, dst, dinv)
    q2 = _prop128(yt, src, dst, dinv)

    out = _tc2(u0, _from_chunks(q1), _from_chunks(q2),
               c2b1.reshape(1, HID), c2b2.reshape(1, HID),
               lw, lb.reshape(1, OUT_DIM))
    return out[:N]


# speedup vs baseline: 8.2183x; 1.3207x over previous
import functools

import jax
import jax.numpy as jnp
from jax import lax
from jax.experimental import pallas as pl
from jax.experimental.pallas import tpu as pltpu
from jax.experimental.pallas import tpu_sc as plsc

N = 10000
NPAD = 10240
E = 320000
EPAD = 327680
EPB = EPAD // 128
IN_DIM = 128
HID = 256
OUT_DIM = 128
NTILES = 16
ROWS_PT = NPAD // NTILES
BN = 256

_MESH = dict(mesh=plsc.VectorSubcoreMesh(core_axis_name="c", subcore_axis_name="s"))


def _rsqrt16(x):
    y = 1.0 / x
    for _ in range(26):
        y = y * (1.5 - 0.5 * x * y * y)
    return y


@functools.partial(
    pl.kernel,
    out_type=(
        jax.ShapeDtypeStruct((NPAD,), jnp.float32),
        jax.ShapeDtypeStruct((NPAD, IN_DIM), jnp.float32),
    ),
    scratch_types=[
        pltpu.VMEM_SHARED((NPAD,), jnp.float32),
        pltpu.VMEM((ROWS_PT,), jnp.float32),
        pltpu.VMEM((128,), jnp.float32),
        pltpu.VMEM((16, 128), jnp.int32),
        pltpu.VMEM((64, IN_DIM), jnp.float32),
        pltpu.VMEM((64, IN_DIM), jnp.float32),
        pltpu.SemaphoreType.DMA,
    ],
    **_MESH,
)
def _sc0(dst_hbm, x_hbm, dinv_hbm, y0_hbm, hist, dvb, ones, dstg, xbuf, ybuf,
         hsem):
    c = lax.axis_index("c")
    s = lax.axis_index("s")

    def fz(i, _):
        dvb[pl.ds(i * 16, 16)] = jnp.zeros((16,), jnp.float32)
        return 0
    lax.fori_loop(0, ROWS_PT // 16, fz, 0)
    for k in range(8):
        ones[pl.ds(k * 16, 16)] = jnp.ones((16,), jnp.float32)
    pltpu.sync_copy(dvb, hist.at[pl.ds(s * ROWS_PT, ROWS_PT)])
    plsc.subcore_barrier()

    base = s * (EPB // NTILES)
    pend = []
    for sb in range(10):
        for d in pend:
            d.wait()
        pend = []
        pltpu.sync_copy(dst_hbm.at[pl.ds(base + sb * 16, 16)], dstg)
        for j in range(16):
            pend.append(
                pltpu.async_copy(ones, hist.at[dstg.at[j]], hsem, add=True))
    for d in pend:
        d.wait()
    plsc.subcore_barrier()

    pltpu.sync_copy(hist.at[pl.ds(s * ROWS_PT, ROWS_PT)], dvb)

    def fb(j, _):
        d = dvb[pl.ds(j * 16, 16)] + 1.0
        dvb[pl.ds(j * 16, 16)] = _rsqrt16(d)
        return 0
    lax.fori_loop(0, ROWS_PT // 16, fb, 0)

    @pl.when(c == 0)
    def _():
        pltpu.sync_copy(dvb, dinv_hbm.at[pl.ds(s * ROWS_PT, ROWS_PT)])

    def fc(i, _):
        r = s * ROWS_PT + c * 320 + i * 64
        pltpu.sync_copy(x_hbm.at[pl.ds(r, 64)], xbuf)
        for j16 in range(4):
            dv16 = dvb[pl.ds(c * 320 + i * 64 + j16 * 16, 16)]
            for l in range(16):
                j = j16 * 16 + l
                dv = dv16[l]
                for k in range(IN_DIM // 16):
                    ybuf[j, pl.ds(k * 16, 16)] = xbuf[j, pl.ds(k * 16, 16)] * dv
        pltpu.sync_copy(ybuf, y0_hbm.at[pl.ds(r, 64)])
        return 0
    lax.fori_loop(0, 5, fc, 0)


@functools.partial(
    pl.kernel,
    out_type=jax.ShapeDtypeStruct((2, NPAD, 128), jnp.float32),
    scratch_types=[
        pltpu.VMEM_SHARED((NPAD, 128), jnp.float32),
        pltpu.VMEM((16, 128), jnp.int32),
        pltpu.VMEM((16, 128), jnp.int32),
        pltpu.VMEM((128, 128), jnp.float32),
        pltpu.VMEM((128, 128), jnp.float32),
        pltpu.SemaphoreType.DMA,
        pltpu.SemaphoreType.DMA,
        pltpu.SemaphoreType.DMA,
        pltpu.SemaphoreType.DMA,
    ],
    **_MESH,
)
def _prop_part(y_hbm, src_hbm, dst_hbm, out_hbm, acc, sstg, dstg,
               rows0, rows1, g0, g1, s0, s1):
    c = lax.axis_index("c")
    s = lax.axis_index("s")
    rows = (rows0, rows1)
    gsem = (g0, g1)
    ssem = (s0, s1)

    for j in range(32):
        for k in range(8):
            rows0[j, pl.ds(k * 16, 16)] = jnp.zeros((16,), jnp.float32)

    def fzc(i, _):
        pltpu.sync_copy(rows0.at[pl.ds(0, 32)],
                        acc.at[pl.ds(s * ROWS_PT + i * 32, 32)])
        return 0
    lax.fori_loop(0, 20, fzc, 0)
    plsc.subcore_barrier()

    blks_pt = EPB // 32
    base = (c * NTILES + s) * blks_pt
    sc_pend = [None, None]
    for sb in range(blks_pt // 16):
        for r in (0, 1):
            if sc_pend[r] is not None:
                sc_pend[r].wait()
                sc_pend[r] = None
        pltpu.sync_copy(src_hbm.at[pl.ds(base + sb * 16, 16)], sstg)
        pltpu.sync_copy(dst_hbm.at[pl.ds(base + sb * 16, 16)], dstg)
        g_pend = [pltpu.async_copy(y_hbm.at[sstg.at[0]], rows0, g0), None]
        for j in range(16):
            r = j % 2
            if j + 1 < 16:
                if sc_pend[1 - r] is not None:
                    sc_pend[1 - r].wait()
                    sc_pend[1 - r] = None
                g_pend[1 - r] = pltpu.async_copy(
                    y_hbm.at[sstg.at[j + 1]], rows[1 - r], gsem[1 - r])
            g_pend[r].wait()
            sc_pend[r] = pltpu.async_copy(
                rows[r], acc.at[dstg.at[j]], ssem[r], add=True)
    for r in (0, 1):
        if sc_pend[r] is not None:
            sc_pend[r].wait()
    plsc.subcore_barrier()

    def fo(i, _):
        r = s * ROWS_PT + i * 64
        pltpu.sync_copy(acc.at[pl.ds(r, 64)], rows0.at[pl.ds(0, 64)])
        pltpu.sync_copy(rows0.at[pl.ds(0, 64)], out_hbm.at[c, pl.ds(r, 64)])
        return 0
    lax.fori_loop(0, 10, fo, 0)


def _make_prop(emit_y):
    one = jax.ShapeDtypeStruct((2 * NPAD, 128), jnp.float32)

    @functools.partial(
        pl.kernel,
        out_type=(one, one) if emit_y else one,
        scratch_types=[
            pltpu.VMEM_SHARED((NPAD, 128), jnp.float32),
            pltpu.VMEM((16, 128), jnp.int32),
            pltpu.VMEM((16, 128), jnp.int32),
            pltpu.VMEM((128, 128), jnp.float32),
            pltpu.VMEM((128, 128), jnp.float32),
            pltpu.VMEM((ROWS_PT,), jnp.float32),
            pltpu.SemaphoreType.DMA,
            pltpu.SemaphoreType.DMA,
            pltpu.SemaphoreType.DMA,
            pltpu.SemaphoreType.DMA,
        ],
        **_MESH,
    )
    def prop(y_hbm, src_hbm, src1_hbm, dst_hbm, dinv_hbm, *rest):
        if emit_y:
            out_hbm, yout_hbm = rest[0], rest[1]
            rest = rest[2:]
        else:
            out_hbm, yout_hbm = rest[0], None
            rest = rest[1:]
        acc, sstg, dstg, rows0, rows1, dvb, g0, g1, s0, s1 = rest
        rows = (rows0, rows1)
        gsem = (g0, g1)
        ssem = (s0, s1)

        c = lax.axis_index("c")
        s = lax.axis_index("s")
        coff = c * NPAD

        for j in range(32):
            for k in range(8):
                rows0[j, pl.ds(k * 16, 16)] = jnp.zeros((16,), jnp.float32)

        def fzc(i, _):
            pltpu.sync_copy(rows0.at[pl.ds(0, 32)],
                            acc.at[pl.ds(s * ROWS_PT + i * 32, 32)])
            return 0
        lax.fori_loop(0, 20, fzc, 0)
        plsc.subcore_barrier()

        blks_pt = EPB // NTILES
        base = s * blks_pt
        sc_pend = [None, None]
        for sb in range(blks_pt // 16):
            for r in (0, 1):
                if sc_pend[r] is not None:
                    sc_pend[r].wait()
                    sc_pend[r] = None

            @pl.when(c == 0)
            def _():
                pltpu.sync_copy(src_hbm.at[pl.ds(base + sb * 16, 16)], sstg)

            @pl.when(c == 1)
            def _():
                pltpu.sync_copy(src1_hbm.at[pl.ds(base + sb * 16, 16)], sstg)

            pltpu.sync_copy(dst_hbm.at[pl.ds(base + sb * 16, 16)], dstg)
            g_pend = [pltpu.async_copy(y_hbm.at[sstg.at[0]], rows0, g0), None]
            for j in range(16):
                r = j % 2
                if j + 1 < 16:
                    if sc_pend[1 - r] is not None:
                        sc_pend[1 - r].wait()
                        sc_pend[1 - r] = None
                    g_pend[1 - r] = pltpu.async_copy(
                        y_hbm.at[sstg.at[j + 1]], rows[1 - r], gsem[1 - r])
                g_pend[r].wait()
                sc_pend[r] = pltpu.async_copy(
                    rows[r], acc.at[dstg.at[j]], ssem[r], add=True)
        for r in (0, 1):
            if sc_pend[r] is not None:
                sc_pend[r].wait()
        plsc.subcore_barrier()

        pltpu.sync_copy(dinv_hbm.at[pl.ds(s * ROWS_PT, ROWS_PT)], dvb)

        def fo(i, _):
            r = s * ROWS_PT + i * 32
            pltpu.sync_copy(acc.at[pl.ds(r, 32)], rows0.at[pl.ds(0, 32)])
            pltpu.sync_copy(y_hbm.at[pl.ds(coff + r, 32)],
                            rows0.at[pl.ds(32, 32)])
            for j16 in range(2):
                dv16 = dvb[pl.ds(i * 32 + j16 * 16, 16)]
                for l in range(16):
                    j = j16 * 16 + l
                    dv = dv16[l]
                    for k in range(8):
                        o = (rows0[j, pl.ds(k * 16, 16)]
                             + rows0[32 + j, pl.ds(k * 16, 16)]) * dv
                        rows0[64 + j, pl.ds(k * 16, 16)] = o
                        if emit_y:
                            rows0[96 + j, pl.ds(k * 16, 16)] = o * dv
            pltpu.sync_copy(rows0.at[pl.ds(64, 32)],
                            out_hbm.at[pl.ds(coff + r, 32)])
            if emit_y:
                pltpu.sync_copy(rows0.at[pl.ds(96, 32)],
                                yout_hbm.at[pl.ds(coff + r, 32)])
            return 0
        lax.fori_loop(0, 20, fo, 0)

    return prop


_prop128y = _make_prop(True)
_prop128 = _make_prop(False)


def _combine_body(pa_ref, pb_ref, y_ref, dv_ref, p_ref, yn_ref):
    dv = dv_ref[...][:, 0:1]
    p = dv * (pa_ref[...] + pb_ref[...] + y_ref[...])
    p_ref[...] = p
    yn_ref[...] = dv * p


def _tc1_body(x_ref, p1_ref, p2_ref, dv_ref, w0, b0, w1, b1, w2, b2,
              wc0, bc0, wc1, wc2, u0_ref, yu1_ref, yu2_ref):
    f32 = jnp.float32
    h0 = jnp.dot(x_ref[...], w0[...], preferred_element_type=f32) + b0[...]
    h1 = jnp.dot(p1_ref[...], w1[...], preferred_element_type=f32) + b1[...]
    h2 = jnp.dot(p2_ref[...], w2[...], preferred_element_type=f32) + b2[...]
    hb = jax.nn.relu(jnp.concatenate([h0, h1, h2], axis=-1))
    dv = dv_ref[...][:, 0:1]
    u0_ref[...] = jnp.dot(hb, wc0[...], preferred_element_type=f32) + bc0[...]
    yu1_ref[...] = jnp.dot(hb, wc1[...], preferred_element_type=f32) * dv
    yu2_ref[...] = jnp.dot(hb, wc2[...], preferred_element_type=f32) * dv


def _tc2_body(u0_ref, q1_ref, q2_ref, bc1, bc2, lw, lb, o_ref):
    h2 = jax.nn.relu(jnp.concatenate(
        [u0_ref[...], q1_ref[...] + bc1[...], q2_ref[...] + bc2[...]], axis=-1))
    o_ref[...] = jnp.dot(h2, lw[...], preferred_element_type=jnp.float32) + lb[...]


def _full(shape):
    return pl.BlockSpec(shape, lambda i: tuple(0 for _ in shape))


def _rows(d):
    return pl.BlockSpec((BN, d), lambda i: (i, 0))


_combine = pl.pallas_call(
    _combine_body,
    grid=(NPAD // BN,),
    in_specs=[_rows(128), _rows(128), _rows(128), _rows(128)],
    out_specs=[_rows(128), _rows(128)],
    out_shape=[
        jax.ShapeDtypeStruct((NPAD, 128), jnp.float32),
        jax.ShapeDtypeStruct((NPAD, 128), jnp.float32),
    ],
)

_tc1 = pl.pallas_call(
    _tc1_body,
    grid=(NPAD // BN,),
    in_specs=[
        _rows(128), _rows(128), _rows(128), _rows(128),
        _full((IN_DIM, HID)), _full((1, HID)),
        _full((IN_DIM, HID)), _full((1, HID)),
        _full((IN_DIM, HID)), _full((1, HID)),
        _full((3 * HID, HID)), _full((1, HID)),
        _full((3 * HID, HID)), _full((3 * HID, HID)),
    ],
    out_specs=[_rows(HID), _rows(HID), _rows(HID)],
    out_shape=[
        jax.ShapeDtypeStruct((NPAD, HID), jnp.float32),
        jax.ShapeDtypeStruct((NPAD, HID), jnp.float32),
        jax.ShapeDtypeStruct((NPAD, HID), jnp.float32),
    ],
)

_tc2 = pl.pallas_call(
    _tc2_body,
    grid=(NPAD // BN,),
    in_specs=[
        _rows(HID), _rows(HID), _rows(HID),
        _full((1, HID)), _full((1, HID)),
        _full((3 * HID, OUT_DIM)), _full((1, OUT_DIM)),
    ],
    out_specs=_rows(OUT_DIM),
    out_shape=jax.ShapeDtypeStruct((NPAD, OUT_DIM), jnp.float32),
)


def _to_chunks(a):
    return a.reshape(NPAD, 2, 128).transpose(1, 0, 2).reshape(2 * NPAD, 128)


def _from_chunks(a):
    return a.reshape(2, NPAD, 128).transpose(1, 0, 2).reshape(NPAD, 256)


def kernel(x, edge_index, c1w0, c1b0, c1w1, c1b1, c1w2, c1b2,
           c2w0, c2b0, c2w1, c2b1, c2w2, c2b2, lw, lb):
    xp = jnp.pad(x, ((0, NPAD - N), (0, 0)))
    src = jnp.pad(edge_index[0].astype(jnp.int32), (0, EPAD - E)).reshape(EPB, 128)
    dst = jnp.pad(edge_index[1].astype(jnp.int32), (0, EPAD - E),
                  constant_values=N).reshape(EPB, 128)

    src1 = src + NPAD

    dinv, y0 = _sc0(dst, xp)
    dv2d = jnp.broadcast_to(dinv[:, None], (NPAD, 128))

    pp = _prop_part(y0, src, dst)
    p1, y1 = _combine(pp[0], pp[1], y0, dv2d)
    pp2 = _prop_part(y1, src, dst)
    p2, _ = _combine(pp2[0], pp2[1], y1, dv2d)

    u0, yu1, yu2 = _tc1(
        xp, p1, p2, dv2d,
        c1w0, c1b0.reshape(1, HID), c1w1, c1b1.reshape(1, HID),
        c1w2, c1b2.reshape(1, HID),
        c2w0, c2b0.reshape(1, HID), c2w1, c2w2,
    )

    q1 = _prop128(_to_chunks(yu1), src, src1, dst, dinv)
    _t, yt = _prop128y(_to_chunks(yu2), src, src1, dst, dinv)
    q2 = _prop128(yt, src, src1, dst, dinv)

    out = _tc2(u0, _from_chunks(q1), _from_chunks(q2),
               c2b1.reshape(1, HID), c2b2.reshape(1, HID),
               lw, lb.reshape(1, OUT_DIM))
    return out[:N]
